# Initial kernel scaffold; baseline (speedup 1.0000x reference)
import jax
import jax.numpy as jnp
from jax.experimental import pallas as pl


def kernel(x, emb_table, W, b):
    raise NotImplementedError("write your pallas kernel here")


# speedup vs baseline: 29.2754x; 29.2754x over previous
import functools

import jax
import jax.numpy as jnp
from jax import lax
from jax.experimental import pallas as pl
from jax.experimental.pallas import tpu as pltpu
from jax.experimental.pallas import tpu_sc as plsc

_B, _L, _D, _V = 4096, 50, 128, 1000
_NC, _NS, _LANES = 2, 16, 16
_NW = _NC * _NS
_ROWS_W = _B // _NW
_IDX_W = _ROWS_W * _L


def _table_body(emb_ref, w_ref, b_ref, out_ref):
    s = jnp.sum(emb_ref[...] * w_ref[...], axis=1, keepdims=True)
    out_ref[...] = (s + b_ref[0, 0]) * (1.0 / _L)


def _fold_table(emb_table, W, b):
    return pl.pallas_call(
        _table_body,
        out_shape=jax.ShapeDtypeStruct((_V, 1), jnp.float32),
    )(emb_table, W, b.reshape(1, 1))


def _sc_body(x_hbm, t_hbm, out_hbm, x_v, t_v, o_v):
    wid = lax.axis_index("s") * _NC + lax.axis_index("c")
    pltpu.sync_copy(x_hbm.at[pl.ds(wid * _IDX_W, _IDX_W)], x_v)
    pltpu.sync_copy(t_hbm, t_v)
    lane = lax.iota(jnp.int32, _LANES)
    for g in range(_ROWS_W // _LANES):
        def body(l, acc, base=g * _LANES * _L):
            off = base + lane * _L + l
            xi = plsc.load_gather(x_v, [off])
            return acc + plsc.load_gather(t_v, [xi])
        acc = lax.fori_loop(0, _L, body, jnp.zeros((_LANES,), jnp.float32))
        o_v[pl.ds(g * _LANES, _LANES)] = 1.0 / (1.0 + jnp.exp(-acc))
    pltpu.sync_copy(o_v, out_hbm.at[pl.ds(wid * _ROWS_W, _ROWS_W)])


@functools.cache
def _sc_call():
    return pl.kernel(
        _sc_body,
        out_type=jax.ShapeDtypeStruct((_B,), jnp.float32),
        mesh=plsc.VectorSubcoreMesh(
            core_axis_name="c", subcore_axis_name="s",
            num_cores=_NC, num_subcores=_NS,
        ),
        scratch_types=[
            pltpu.VMEM((_IDX_W,), jnp.int32),
            pltpu.VMEM((_V,), jnp.float32),
            pltpu.VMEM((_ROWS_W,), jnp.float32),
        ],
        compiler_params=pltpu.CompilerParams(needs_layout_passes=False),
    )


def kernel(x, emb_table, W, b):
    t = _fold_table(emb_table, W, b).reshape(_V)
    out = _sc_call()(x.reshape(_B * _L), t)
    return out.reshape(_B, 1)


# speedup vs baseline: 30.2564x; 1.0335x over previous
import functools

import jax
import jax.numpy as jnp
from jax import lax
from jax.experimental import pallas as pl
from jax.experimental.pallas import tpu as pltpu
from jax.experimental.pallas import tpu_sc as plsc

_B, _L, _D, _V = 4096, 50, 128, 1000
_NC, _NS, _LANES = 2, 16, 16
_NW = _NC * _NS
_ROWS_W = _B // _NW
_IDX_W = _ROWS_W * _L


def _table_body(emb_ref, w_ref, b_ref, out_ref):
    s = jnp.sum(emb_ref[...] * w_ref[...], axis=1, keepdims=True)
    out_ref[...] = (s + b_ref[0, 0]) * (1.0 / _L)


def _fold_table(emb_table, W, b):
    return pl.pallas_call(
        _table_body,
        out_shape=jax.ShapeDtypeStruct((_V, 1), jnp.float32),
    )(emb_table, W, b.reshape(1, 1))


def _sc_body(x_hbm, t_hbm, out_hbm, x_v, t_v, o_v):
    wid = lax.axis_index("s") * _NC + lax.axis_index("c")
    pltpu.sync_copy(x_hbm.at[pl.ds(wid * _IDX_W, _IDX_W)], x_v)
    pltpu.sync_copy(t_hbm, t_v)
    lane = lax.iota(jnp.int32, _LANES)
    row_off = lane * _L
    for g in range(_ROWS_W // _LANES):
        accs = [jnp.zeros((_LANES,), jnp.float32) for _ in range(4)]
        base = g * _LANES * _L
        for l in range(_L):
            xi = plsc.load_gather(x_v, [row_off + (base + l)])
            accs[l % 4] = accs[l % 4] + plsc.load_gather(t_v, [xi])
        acc = (accs[0] + accs[1]) + (accs[2] + accs[3])
        o_v[pl.ds(g * _LANES, _LANES)] = 1.0 / (1.0 + jnp.exp(-acc))
    pltpu.sync_copy(o_v, out_hbm.at[pl.ds(wid * _ROWS_W, _ROWS_W)])


@functools.cache
def _sc_call():
    return pl.kernel(
        _sc_body,
        out_type=jax.ShapeDtypeStruct((_B,), jnp.float32),
        mesh=plsc.VectorSubcoreMesh(
            core_axis_name="c", subcore_axis_name="s",
            num_cores=_NC, num_subcores=_NS,
        ),
        scratch_types=[
            pltpu.VMEM((_IDX_W,), jnp.int32),
            pltpu.VMEM((_V,), jnp.float32),
            pltpu.VMEM((_ROWS_W,), jnp.float32),
        ],
        compiler_params=pltpu.CompilerParams(needs_layout_passes=False),
    )


def kernel(x, emb_table, W, b):
    t = _fold_table(emb_table, W, b).reshape(_V)
    out = _sc_call()(x.reshape(_B * _L), t)
    return out.reshape(_B, 1)
